# baseline (speedup 1.0000x reference)
import functools

import jax
import jax.numpy as jnp
from jax import lax
from jax.experimental import pallas as pl
from jax.experimental.pallas import tpu as pltpu
from jax.experimental.pallas import tpu_sc as plsc

BATCH = 16384
D_MODEL = 512
LANES = 16

NUM_CORES = 2
NUM_SUBCORES = 16
NUM_WORKERS = NUM_CORES * NUM_SUBCORES
B_PER_W = BATCH // NUM_WORKERS
CHUNK = 16
NCHUNKS = B_PER_W // CHUNK
NB = 4
NROUNDS = NCHUNKS // NB


def _body(x_hbm, idx_hbm, tbl_hbm, out_hbm,
          idx_v, xbuf, rbuf, gsem, xsem, ssem):
    wid = lax.axis_index("s") * NUM_CORES + lax.axis_index("c")
    base = wid * B_PER_W
    pltpu.sync_copy(idx_hbm.at[pl.ds(base, B_PER_W)], idx_v)

    def issue_gather(q, b):
        pltpu.async_copy(
            tbl_hbm.at[idx_v.at[pl.ds(q * CHUNK, CHUNK)]],
            rbuf.at[b], gsem.at[b])

    def issue_xload(q, b):
        pltpu.async_copy(
            x_hbm.at[pl.ds(base + q * CHUNK, CHUNK)],
            xbuf.at[b], xsem.at[b])

    def issue_store(q, b):
        pltpu.async_copy(
            xbuf.at[b], out_hbm.at[pl.ds(base + q * CHUNK, CHUNK)],
            ssem.at[b])

    def wait_gather(b):
        pltpu.make_async_copy(
            tbl_hbm.at[pl.ds(0, CHUNK)], rbuf.at[b], gsem.at[b]).wait()

    def wait_xload(b):
        pltpu.make_async_copy(
            x_hbm.at[pl.ds(0, CHUNK)], xbuf.at[b], xsem.at[b]).wait()

    def wait_store(b):
        pltpu.make_async_copy(
            xbuf.at[b], out_hbm.at[pl.ds(0, CHUNK)], ssem.at[b]).wait()

    def add_chunk(b):
        @plsc.parallel_loop(0, CHUNK, step=1, unroll=2)
        def add_row(i):
            for j in range(D_MODEL // LANES):
                sl = pl.ds(j * LANES, LANES)
                plsc.addupdate(xbuf.at[b, i, sl], rbuf[b, i, sl])

    def step(q, b, do_gather, do_xload, wait_prev_store):
        wait_gather(b)
        wait_xload(b)
        add_chunk(b)
        issue_store(q, b)
        if do_gather:
            issue_gather(q + NB, b)
        if do_xload:
            b2 = (b + 2) % NB
            if wait_prev_store:
                wait_store(b2)
            issue_xload(q + 2, b2)

    for b in range(NB):
        issue_gather(b, b)
    for b in range(2):
        issue_xload(b, b)

    for b in range(NB):
        step(b, b, True, True, b >= 2)

    def round_body(r, _):
        for b in range(NB):
            step(r * NB + b, b, True, True, True)
        return 0

    lax.fori_loop(1, NROUNDS - 1, round_body, 0)

    for b in range(NB):
        step((NROUNDS - 1) * NB + b, b, False, b < 2, b < 2)
    for b in range(NB):
        wait_store(b)


def kernel(x, context_info, context_emb_weight):
    mesh = plsc.VectorSubcoreMesh(core_axis_name="c", subcore_axis_name="s")
    kfn = functools.partial(
        pl.kernel,
        mesh=mesh,
        out_type=jax.ShapeDtypeStruct((BATCH, D_MODEL), jnp.float32),
        scratch_types=[
            pltpu.VMEM((B_PER_W,), jnp.int32),
            pltpu.VMEM((NB, CHUNK, D_MODEL), jnp.float32),
            pltpu.VMEM((NB, CHUNK, D_MODEL), jnp.float32),
            pltpu.SemaphoreType.DMA((NB,)),
            pltpu.SemaphoreType.DMA((NB,)),
            pltpu.SemaphoreType.DMA((NB,)),
        ],
    )(_body)
    return kfn(x, context_info.astype(jnp.int32), context_emb_weight)


# speedup vs baseline: 1.0546x; 1.0546x over previous
import functools

import jax
import jax.numpy as jnp
from jax import lax
from jax.experimental import pallas as pl
from jax.experimental.pallas import tpu as pltpu
from jax.experimental.pallas import tpu_sc as plsc

BATCH = 16384
D_MODEL = 512
LANES = 16

NUM_CORES = 2
NUM_SUBCORES = 16
NUM_WORKERS = NUM_CORES * NUM_SUBCORES
B_PER_W = BATCH // NUM_WORKERS
CHUNK = 32
NCHUNKS = B_PER_W // CHUNK
NG = 3
XCHUNK = 2 * CHUNK
NPAIRS = B_PER_W // XCHUNK
NX = 2


def _body(x_hbm, idx_hbm, tbl_hbm, out_hbm,
          idx_v, xbuf, rbuf, gsem, xsem, ssem):
    wid = lax.axis_index("s") * NUM_CORES + lax.axis_index("c")
    base = wid * B_PER_W
    pltpu.sync_copy(idx_hbm.at[pl.ds(base, B_PER_W)], idx_v)

    def issue_gather(c):
        return pltpu.async_copy(
            tbl_hbm.at[idx_v.at[pl.ds(c * CHUNK, CHUNK)]],
            rbuf.at[c % NG], gsem.at[c % NG])

    def issue_xload(k):
        return pltpu.async_copy(
            x_hbm.at[pl.ds(base + k * XCHUNK, XCHUNK)],
            xbuf.at[k % NX], xsem.at[k % NX])

    def issue_store(k):
        return pltpu.async_copy(
            xbuf.at[k % NX], out_hbm.at[pl.ds(base + k * XCHUNK, XCHUNK)],
            ssem.at[k % NX])

    gathers = {c: issue_gather(c) for c in range(NG)}
    xloads = {k: issue_xload(k) for k in range(NX)}
    stores = {}

    for c in range(NCHUNKS):
        k, h, s, rg = c // 2, c % 2, (c // 2) % NX, c % NG
        gathers.pop(c).wait()
        if h == 0:
            xloads.pop(k).wait()
            if 1 <= k <= NPAIRS - 2:
                stores.pop(k - 1).wait()
                xloads[k + 1] = issue_xload(k + 1)

        @plsc.parallel_loop(0, CHUNK, step=1, unroll=2 if h == 0 else 1)
        def add_row(i):
            for j in range(D_MODEL // LANES):
                sl = pl.ds(j * LANES, LANES)
                plsc.addupdate(xbuf.at[s, h * CHUNK + i, sl],
                               rbuf[rg, i, sl])

        if h == 1:
            stores[k] = issue_store(k)
        if c + NG < NCHUNKS:
            gathers[c + NG] = issue_gather(c + NG)
    for k in sorted(stores):
        stores.pop(k).wait()


def kernel(x, context_info, context_emb_weight):
    mesh = plsc.VectorSubcoreMesh(core_axis_name="c", subcore_axis_name="s")
    kfn = functools.partial(
        pl.kernel,
        mesh=mesh,
        out_type=jax.ShapeDtypeStruct((BATCH, D_MODEL), jnp.float32),
        scratch_types=[
            pltpu.VMEM((B_PER_W,), jnp.int32),
            pltpu.VMEM((NX, XCHUNK, D_MODEL), jnp.float32),
            pltpu.VMEM((NG, CHUNK, D_MODEL), jnp.float32),
            pltpu.SemaphoreType.DMA((NG,)),
            pltpu.SemaphoreType.DMA((NX,)),
            pltpu.SemaphoreType.DMA((NX,)),
        ],
    )(_body)
    return kfn(x, context_info.astype(jnp.int32), context_emb_weight)
